# baseline (speedup 1.0000x reference)
import functools

import jax
import jax.numpy as jnp
from jax import lax
from jax.experimental import pallas as pl
from jax.experimental.pallas import tpu as pltpu
from jax.experimental.pallas import tpu_sc as plsc

N = 10000
NPAD = 10240
E = 320000
D = 128
H = 64
C = 10
G = 64

NC = 2
NS = 16
LW = 128
IW = 256
NB = 40
NR = 4
EP = NC * NS * IW * NB
ROWS_T = NPAD // NS

_sc_mesh = plsc.VectorSubcoreMesh(core_axis_name="c", subcore_axis_name="s")
_sc_params = pltpu.CompilerParams(use_tc_tiling_on_sc=False)


@functools.partial(
    pl.kernel,
    out_type=jax.ShapeDtypeStruct((NC * NPAD, 8), jnp.float32),
    mesh=_sc_mesh,
    compiler_params=_sc_params,
    scratch_types=[
        pltpu.VMEM_SHARED((NPAD, 8), jnp.float32),
        pltpu.VMEM((NB, IW), jnp.int32),
        pltpu.VMEM((IW, 8), jnp.float32),
    ],
)
def _sc_degree(dst_hbm, zeros8_hbm, ones_hbm, out_hbm, acc, dstv, onesv):
    cid = lax.axis_index("c")
    sid = lax.axis_index("s")
    wid = cid * NS + sid

    pltpu.sync_copy(zeros8_hbm, acc.at[pl.ds(sid * ROWS_T, ROWS_T), :])
    pltpu.sync_copy(dst_hbm.at[pl.ds(wid * NB, NB), :], dstv)
    pltpu.sync_copy(ones_hbm, onesv)
    plsc.subcore_barrier()

    def body(j, _):
        pltpu.sync_copy(onesv, acc.at[dstv.at[j]], add=True)
        return _

    lax.fori_loop(0, NB, body, None)
    plsc.subcore_barrier()

    pltpu.sync_copy(
        acc.at[pl.ds(sid * ROWS_T, ROWS_T), :],
        out_hbm.at[pl.ds(cid * NPAD + sid * ROWS_T, ROWS_T), :],
    )


@functools.partial(
    pl.kernel,
    out_type=jax.ShapeDtypeStruct((NC * NPAD, H), jnp.float32),
    mesh=_sc_mesh,
    compiler_params=_sc_params,
    scratch_types=[
        pltpu.VMEM_SHARED((NPAD, H), jnp.float32),
        pltpu.VMEM((NB, IW), jnp.int32),
        pltpu.VMEM((NB, IW), jnp.int32),
        [pltpu.VMEM((IW, H), jnp.float32)] * NR,
        [pltpu.SemaphoreType.DMA] * NR,
        [pltpu.SemaphoreType.DMA] * NR,
    ],
)
def _sc_edge_agg(src_hbm, dst_hbm, g_hbm, zeros_hbm, out_hbm,
                 acc, srcv, dstv, rows, gsem, ssem):
    cid = lax.axis_index("c")
    sid = lax.axis_index("s")
    wid = cid * NS + sid
    nstep = NB

    pltpu.sync_copy(zeros_hbm, acc.at[pl.ds(sid * ROWS_T, ROWS_T), :])
    pltpu.sync_copy(src_hbm.at[pl.ds(wid * NB, NB), :], srcv)
    pltpu.sync_copy(dst_hbm.at[pl.ds(wid * NB, NB), :], dstv)
    plsc.subcore_barrier()

    def gather_start(j, k):
        pltpu.async_copy(g_hbm.at[srcv.at[j]], rows[k], gsem[k])

    def gather_wait(j, k):
        pltpu.make_async_copy(g_hbm.at[srcv.at[j]], rows[k], gsem[k]).wait()

    def scatter_start(j, k):
        pltpu.async_copy(rows[k], acc.at[dstv.at[j]], ssem[k], add=True)

    def scatter_wait(j, k):
        pltpu.make_async_copy(rows[k], acc.at[dstv.at[j]], ssem[k]).wait()

    for k in range(NR - 1):
        gather_start(k, k)

    def body(j, _):
        for k in range(NR):
            @pl.when(j % NR == k)
            def _(k=k):
                gather_wait(j, k)
                kn = (k + NR - 1) % NR

                @pl.when(j + NR - 1 < nstep)
                def _():
                    @pl.when(j >= 1)
                    def _():
                        scatter_wait(j - 1, kn)

                    gather_start(j + NR - 1, kn)

                scatter_start(j, k)

        return _

    lax.fori_loop(0, nstep, body, None)
    for t in range(NR, 0, -1):
        scatter_wait(nstep - t, (nstep - t) % NR)
    plsc.subcore_barrier()

    pltpu.sync_copy(
        acc.at[pl.ds(sid * ROWS_T, ROWS_T), :],
        out_hbm.at[pl.ds(cid * NPAD + sid * ROWS_T, ROWS_T), :],
    )


def _tc_dense1_body(x_ref, w1_ref, degp_ref, g_ref, dis_ref):
    deg = (degp_ref[0:NPAD, 0:1] + degp_ref[NPAD:2 * NPAD, 0:1]) + 1.0
    dis = lax.rsqrt(deg)
    h = jnp.dot(x_ref[...], w1_ref[...], preferred_element_type=jnp.float32)
    g_ref[...] = h * dis
    dis_ref[...] = dis


def _tc_dense2_body(sp_ref, g1_ref, dis_ref, b1_ref, w2_ref, g2_ref):
    dis = dis_ref[...]
    s = sp_ref[0:NPAD, :] + sp_ref[NPAD:2 * NPAD, :]
    h1 = jnp.maximum(dis * (s + g1_ref[...]) + b1_ref[...], 0.0)
    g2_ref[...] = jnp.dot(h1, w2_ref[...], preferred_element_type=jnp.float32) * dis


def _tc_final_body(sp_ref, g2_ref, dis_ref, b2_ref, batch_ref, wfc_ref, bfc_ref,
                   out_ref):
    dis = dis_ref[...]
    s = sp_ref[0:NPAD, :] + sp_ref[NPAD:2 * NPAD, :]
    h2 = jnp.maximum(dis * (s + g2_ref[...]) + b2_ref[...], 0.0)
    gid = lax.broadcasted_iota(jnp.int32, (1, G), 1)
    onehot = (batch_ref[...] == gid).astype(jnp.float32)
    sums = lax.dot_general(onehot, h2, (((0,), (0,)), ((), ())),
                           preferred_element_type=jnp.float32)
    counts = jnp.sum(onehot, axis=0)[:, None]
    pooled = sums / jnp.maximum(counts, 1.0)
    out_ref[...] = (jnp.dot(pooled, wfc_ref[...], preferred_element_type=jnp.float32)
                    + bfc_ref[...])


_tc_dense1 = pl.pallas_call(
    _tc_dense1_body,
    out_shape=(jax.ShapeDtypeStruct((NPAD, H), jnp.float32),
               jax.ShapeDtypeStruct((NPAD, 1), jnp.float32)),
)

_tc_dense2 = pl.pallas_call(
    _tc_dense2_body,
    out_shape=jax.ShapeDtypeStruct((NPAD, H), jnp.float32),
)

_tc_final = pl.pallas_call(
    _tc_final_body,
    out_shape=jax.ShapeDtypeStruct((G, C), jnp.float32),
)


def kernel(x, edge_index, batch, W1, b1, W2, b2, Wfc, bfc):
    src = edge_index[0].astype(jnp.int32)
    dst = edge_index[1].astype(jnp.int32)
    pad = N + (jnp.arange(EP - E, dtype=jnp.int32) % (NPAD - N))
    src_p = jnp.concatenate([src, pad]).reshape(EP // IW, IW)
    dst_p = jnp.concatenate([dst, pad]).reshape(EP // IW, IW)

    x_p = jnp.pad(x, ((0, NPAD - N), (0, 0)))
    batch_p = jnp.concatenate(
        [batch.astype(jnp.int32), jnp.full((NPAD - N,), G, jnp.int32)]
    ).reshape(NPAD, 1)

    zeros_rows = jnp.zeros((ROWS_T, H), jnp.float32)
    zeros8 = jnp.zeros((ROWS_T, 8), jnp.float32)
    ones_row = jnp.ones((IW, 8), jnp.float32)

    degp = _sc_degree(dst_p, zeros8, ones_row)
    g1, dis = _tc_dense1(x_p, W1, degp)
    s1 = _sc_edge_agg(src_p, dst_p, g1, zeros_rows)
    g2 = _tc_dense2(s1, g1, dis, b1.reshape(1, H), W2)
    s2 = _sc_edge_agg(src_p, dst_p, g2, zeros_rows)
    return _tc_final(s2, g2, dis, b2.reshape(1, H), batch_p, Wfc,
                     bfc.reshape(1, C))


# speedup vs baseline: 1.0056x; 1.0056x over previous
import functools

import jax
import jax.numpy as jnp
from jax import lax
from jax.experimental import pallas as pl
from jax.experimental.pallas import tpu as pltpu
from jax.experimental.pallas import tpu_sc as plsc

N = 10000
NPAD = 10240
E = 320000
D = 128
H = 64
C = 10
G = 64

NC = 2
NS = 16
LW = 128
IW = 320
NB = 32
NR = 3
EP = NC * NS * IW * NB
ROWS_T = NPAD // NS

_sc_mesh = plsc.VectorSubcoreMesh(core_axis_name="c", subcore_axis_name="s")
_sc_params = pltpu.CompilerParams(use_tc_tiling_on_sc=False)


@functools.partial(
    pl.kernel,
    out_type=jax.ShapeDtypeStruct((NC * NPAD, 8), jnp.float32),
    mesh=_sc_mesh,
    compiler_params=_sc_params,
    scratch_types=[
        pltpu.VMEM_SHARED((NPAD, 8), jnp.float32),
        pltpu.VMEM((NB, IW), jnp.int32),
        pltpu.VMEM((IW, 8), jnp.float32),
    ],
)
def _sc_degree(dst_hbm, zeros8_hbm, ones_hbm, out_hbm, acc, dstv, onesv):
    cid = lax.axis_index("c")
    sid = lax.axis_index("s")
    wid = cid * NS + sid

    pltpu.sync_copy(zeros8_hbm, acc.at[pl.ds(sid * ROWS_T, ROWS_T), :])
    pltpu.sync_copy(dst_hbm.at[pl.ds(wid * NB, NB), :], dstv)
    pltpu.sync_copy(ones_hbm, onesv)
    plsc.subcore_barrier()

    def body(j, _):
        pltpu.sync_copy(onesv, acc.at[dstv.at[j]], add=True)
        return _

    lax.fori_loop(0, NB, body, None)
    plsc.subcore_barrier()

    pltpu.sync_copy(
        acc.at[pl.ds(sid * ROWS_T, ROWS_T), :],
        out_hbm.at[pl.ds(cid * NPAD + sid * ROWS_T, ROWS_T), :],
    )


@functools.partial(
    pl.kernel,
    out_type=jax.ShapeDtypeStruct((NC * NPAD, H), jnp.float32),
    mesh=_sc_mesh,
    compiler_params=_sc_params,
    scratch_types=[
        pltpu.VMEM_SHARED((NPAD, H), jnp.float32),
        pltpu.VMEM((NB, IW), jnp.int32),
        pltpu.VMEM((NB, IW), jnp.int32),
        [pltpu.VMEM((IW, H), jnp.float32)] * NR,
        [pltpu.SemaphoreType.DMA] * NR,
        [pltpu.SemaphoreType.DMA] * NR,
    ],
)
def _sc_edge_agg(src_hbm, dst_hbm, g_hbm, zeros_hbm, out_hbm,
                 acc, srcv, dstv, rows, gsem, ssem):
    cid = lax.axis_index("c")
    sid = lax.axis_index("s")
    wid = cid * NS + sid
    nstep = NB

    pltpu.sync_copy(zeros_hbm, acc.at[pl.ds(sid * ROWS_T, ROWS_T), :])
    pltpu.sync_copy(src_hbm.at[pl.ds(wid * NB, NB), :], srcv)
    pltpu.sync_copy(dst_hbm.at[pl.ds(wid * NB, NB), :], dstv)
    plsc.subcore_barrier()

    def gather_start(j, k):
        pltpu.async_copy(g_hbm.at[srcv.at[j]], rows[k], gsem[k])

    def gather_wait(j, k):
        pltpu.make_async_copy(g_hbm.at[srcv.at[j]], rows[k], gsem[k]).wait()

    def scatter_start(j, k):
        pltpu.async_copy(rows[k], acc.at[dstv.at[j]], ssem[k], add=True)

    def scatter_wait(j, k):
        pltpu.make_async_copy(rows[k], acc.at[dstv.at[j]], ssem[k]).wait()

    for k in range(NR - 1):
        gather_start(k, k)

    def body(j, _):
        for k in range(NR):
            @pl.when(j % NR == k)
            def _(k=k):
                gather_wait(j, k)
                kn = (k + NR - 1) % NR

                @pl.when(j + NR - 1 < nstep)
                def _():
                    @pl.when(j >= 1)
                    def _():
                        scatter_wait(j - 1, kn)

                    gather_start(j + NR - 1, kn)

                scatter_start(j, k)

        return _

    lax.fori_loop(0, nstep, body, None)
    for t in range(NR, 0, -1):
        scatter_wait(nstep - t, (nstep - t) % NR)
    plsc.subcore_barrier()

    pltpu.sync_copy(
        acc.at[pl.ds(sid * ROWS_T, ROWS_T), :],
        out_hbm.at[pl.ds(cid * NPAD + sid * ROWS_T, ROWS_T), :],
    )


def _tc_dense1_body(x_ref, w1_ref, degp_ref, g_ref, dis_ref):
    deg = (degp_ref[0:NPAD, 0:1] + degp_ref[NPAD:2 * NPAD, 0:1]) + 1.0
    dis = lax.rsqrt(deg)
    h = jnp.dot(x_ref[...], w1_ref[...], preferred_element_type=jnp.float32)
    g_ref[...] = h * dis
    dis_ref[...] = dis


def _tc_dense2_body(sp_ref, g1_ref, dis_ref, b1_ref, w2_ref, g2_ref):
    dis = dis_ref[...]
    s = sp_ref[0:NPAD, :] + sp_ref[NPAD:2 * NPAD, :]
    h1 = jnp.maximum(dis * (s + g1_ref[...]) + b1_ref[...], 0.0)
    g2_ref[...] = jnp.dot(h1, w2_ref[...], preferred_element_type=jnp.float32) * dis


def _tc_final_body(sp_ref, g2_ref, dis_ref, b2_ref, batch_ref, wfc_ref, bfc_ref,
                   out_ref):
    dis = dis_ref[...]
    s = sp_ref[0:NPAD, :] + sp_ref[NPAD:2 * NPAD, :]
    h2 = jnp.maximum(dis * (s + g2_ref[...]) + b2_ref[...], 0.0)
    gid = lax.broadcasted_iota(jnp.int32, (1, G), 1)
    onehot = (batch_ref[...] == gid).astype(jnp.float32)
    sums = lax.dot_general(onehot, h2, (((0,), (0,)), ((), ())),
                           preferred_element_type=jnp.float32)
    counts = jnp.sum(onehot, axis=0)[:, None]
    pooled = sums / jnp.maximum(counts, 1.0)
    out_ref[...] = (jnp.dot(pooled, wfc_ref[...], preferred_element_type=jnp.float32)
                    + bfc_ref[...])


_tc_dense1 = pl.pallas_call(
    _tc_dense1_body,
    out_shape=(jax.ShapeDtypeStruct((NPAD, H), jnp.float32),
               jax.ShapeDtypeStruct((NPAD, 1), jnp.float32)),
)

_tc_dense2 = pl.pallas_call(
    _tc_dense2_body,
    out_shape=jax.ShapeDtypeStruct((NPAD, H), jnp.float32),
)

_tc_final = pl.pallas_call(
    _tc_final_body,
    out_shape=jax.ShapeDtypeStruct((G, C), jnp.float32),
)


def kernel(x, edge_index, batch, W1, b1, W2, b2, Wfc, bfc):
    src = edge_index[0].astype(jnp.int32)
    dst = edge_index[1].astype(jnp.int32)
    pad = N + (jnp.arange(EP - E, dtype=jnp.int32) % (NPAD - N))
    src_p = jnp.concatenate([src, pad]).reshape(EP // IW, IW)
    dst_p = jnp.concatenate([dst, pad]).reshape(EP // IW, IW)

    x_p = jnp.pad(x, ((0, NPAD - N), (0, 0)))
    batch_p = jnp.concatenate(
        [batch.astype(jnp.int32), jnp.full((NPAD - N,), G, jnp.int32)]
    ).reshape(NPAD, 1)

    zeros_rows = jnp.zeros((ROWS_T, H), jnp.float32)
    zeros8 = jnp.zeros((ROWS_T, 8), jnp.float32)
    ones_row = jnp.ones((IW, 8), jnp.float32)

    degp = _sc_degree(dst_p, zeros8, ones_row)
    g1, dis = _tc_dense1(x_p, W1, degp)
    s1 = _sc_edge_agg(src_p, dst_p, g1, zeros_rows)
    g2 = _tc_dense2(s1, g1, dis, b1.reshape(1, H), W2)
    s2 = _sc_edge_agg(src_p, dst_p, g2, zeros_rows)
    return _tc_final(s2, g2, dis, b2.reshape(1, H), batch_p, Wfc,
                     bfc.reshape(1, C))
